# baseline (speedup 1.0000x reference)
import functools

import jax
import jax.numpy as jnp
from jax.experimental import pallas as pl
from jax.experimental.pallas import tpu as pltpu

N_NODES = 100
N_EDGES = 3200
NP = 128
EPS = 1e-5


def _rsqrt(v):
    r = jax.lax.rsqrt(v)
    r = r * (1.5 - 0.5 * v * r * r)
    r = r * (1.5 - 0.5 * v * r * r)
    return r


def _bn(h, gamma, beta):
    inv_n = 1.0 / N_NODES
    mean = jnp.sum(h, axis=0, keepdims=True) * inv_n
    xc = h - mean
    var = jnp.sum(xc * xc, axis=0, keepdims=True) * inv_n
    return xc * _rsqrt(var + EPS) * gamma + beta


def _gcn_kernel(ei_ref, x_hbm, w1_hbm, b1_hbm, w2_hbm, b2_hbm, g_hbm, be_hbm,
                l1w_hbm, l1b_hbm, l2w_hbm, l2b_hbm, l3w_hbm, l3b_hbm,
                out_ref,
                x_s, w1_s, b1_s, w2_s, b2_s, g_s, be_s, l1w_s, l1b_s, l2w_s,
                l2b_s, l3w_s, l3b_s, sems):
    f32 = jnp.float32
    bf = jnp.bfloat16

    pairs = [(x_hbm, x_s), (w1_hbm, w1_s), (b1_hbm, b1_s), (w2_hbm, w2_s),
             (b2_hbm, b2_s), (g_hbm, g_s), (be_hbm, be_s), (l1w_hbm, l1w_s),
             (l1b_hbm, l1b_s), (l2w_hbm, l2w_s), (l2b_hbm, l2b_s),
             (l3w_hbm, l3w_s), (l3b_hbm, l3b_s)]
    copies = [pltpu.make_async_copy(src, dst, sems.at[i])
              for i, (src, dst) in enumerate(pairs)]
    for c in copies:
        c.start()

    srcv = ei_ref[0:1, :]
    dstv = ei_ref[1:2, :]
    jrow = jax.lax.broadcasted_iota(jnp.int32, (NP, N_EDGES), 0)
    st = (jrow == srcv).astype(bf)
    dt = (jrow == dstv).astype(bf)

    ii = jax.lax.broadcasted_iota(jnp.int32, (NP, NP), 0)
    jj = jax.lax.broadcasted_iota(jnp.int32, (NP, NP), 1)
    eye = ((ii == jj) & (ii < N_NODES)).astype(f32)
    cnt = jax.lax.dot_general(dt, st, (((1,), (1,)), ((), ())),
                              preferred_element_type=f32) + eye
    deg = jnp.sum(cnt, axis=1, keepdims=True)
    dis_c = jnp.where(deg > 0, _rsqrt(jnp.maximum(deg, 1.0)), 0.0)
    dis_r = jnp.sum(jnp.where(ii == jj, dis_c, 0.0), axis=0, keepdims=True)
    a = (cnt * dis_c * dis_r)[:N_NODES, :N_NODES]

    hi = jax.lax.Precision.HIGHEST
    for c in copies[:7]:
        c.wait()

    xw1 = jnp.dot(x_s[...].astype(bf), w1_s[...].astype(bf),
                  preferred_element_type=f32)
    h = jnp.dot(a, xw1, preferred_element_type=f32, precision=hi) + b1_s[...]
    h = _bn(jax.nn.relu(h), g_s[...], be_s[...])

    xw2 = jnp.dot(h.astype(bf), w2_s[...].astype(bf),
                  preferred_element_type=f32)
    h = jnp.dot(a, xw2, preferred_element_type=f32, precision=hi) + b2_s[...]
    h = _bn(jax.nn.relu(h), g_s[...], be_s[...])

    for c in copies[7:]:
        c.wait()
    prod = h.astype(bf).astype(f32)[:, :, None] * l1w_s[...].astype(bf).astype(f32)
    fc1 = jnp.sum(jnp.sum(prod, axis=0), axis=0, keepdims=True)
    r = jax.nn.relu(fc1 + l1b_s[...])
    r = jax.nn.relu(jnp.dot(r.astype(bf), l2w_s[...].astype(bf),
                            preferred_element_type=f32) + l2b_s[...])
    out_ref[...] = (jnp.dot(r.astype(bf), l3w_s[...].astype(bf),
                            preferred_element_type=f32) + l3b_s[...])


@functools.partial(jax.jit, static_argnames=())
def kernel(x, edge_index, W1, b1, W2, b2, gamma, beta, l1W, l1b, l2W, l2b,
           l3W, l3b):
    vmem = pl.BlockSpec(memory_space=pltpu.MemorySpace.VMEM)
    hbm = pl.BlockSpec(memory_space=pltpu.MemorySpace.HBM)
    V = pltpu.MemorySpace.VMEM
    f32 = jnp.float32
    out = pl.pallas_call(
        _gcn_kernel,
        out_shape=jax.ShapeDtypeStruct((1, 2), jnp.float32),
        in_specs=[vmem] + [hbm] * 13,
        out_specs=vmem,
        scratch_shapes=[
            V((N_NODES, 2), f32), V((2, 64), f32), V((1, 64), f32),
            V((64, 64), f32), V((1, 64), f32), V((1, 64), f32),
            V((1, 64), f32), V((N_NODES, 64, 64), f32), V((1, 64), f32),
            V((64, 64), f32), V((1, 64), f32), V((64, 2), f32),
            V((1, 2), f32),
            pltpu.SemaphoreType.DMA((13,)),
        ],
    )(
        edge_index, x, W1, b1.reshape(1, -1), W2, b2.reshape(1, -1),
        gamma.reshape(1, -1), beta.reshape(1, -1),
        l1W.reshape(N_NODES, 64, 64), l1b.reshape(1, -1),
        l2W, l2b.reshape(1, -1), l3W, l3b.reshape(1, -1),
    )
    return out
